# baseline (speedup 1.0000x reference)
import functools

import jax
import jax.numpy as jnp
from jax import lax
from jax.experimental import pallas as pl
from jax.experimental.pallas import tpu as pltpu
from jax.experimental.pallas import tpu_sc as plsc

BATCH = 4
ROWS = 8192
D = 1024

NC = 2
NS = 16
NW = NC * NS
RPW = ROWS // NW
C = 32
NCHUNKS = RPW // C
NBUF = 2

_mesh = plsc.VectorSubcoreMesh(core_axis_name="c", subcore_axis_name="s")


@functools.partial(
    pl.kernel,
    mesh=_mesh,
    out_type=jax.ShapeDtypeStruct((BATCH * ROWS, D), jnp.float32),
    scratch_types=[
        pltpu.VMEM((NBUF, C, D), jnp.float32),
        pltpu.SemaphoreType.DMA,
        pltpu.SemaphoreType.DMA,
    ],
)
def _sc_copy(w_hbm, out_hbm, buf, lsem, ssem):
    wid = lax.axis_index("s") * NC + lax.axis_index("c")
    base = wid * RPW

    def load(ci):
        return pltpu.make_async_copy(
            w_hbm.at[pl.ds(base + ci * C, C)], buf.at[ci % NBUF], lsem)

    def store(ci, b):
        return pltpu.make_async_copy(
            buf.at[ci % NBUF], out_hbm.at[pl.ds(b * ROWS + base + ci * C, C)],
            ssem)

    load(0).start()
    load(1).start()
    for ci in range(NCHUNKS):
        load(ci).wait()
        for b in range(BATCH):
            store(ci, b).start()
        if ci >= 1:
            for b in range(BATCH):
                store(ci - 1, b).wait()
            if ci + 1 < NCHUNKS:
                load(ci + 1).start()
    for b in range(BATCH):
        store(NCHUNKS - 1, b).wait()


def kernel(input_ids, weight):
    del input_ids
    out = _sc_copy(weight)
    return out.reshape(BATCH, ROWS, D)


# speedup vs baseline: 1.0176x; 1.0176x over previous
import functools

import jax
import jax.numpy as jnp
from jax import lax
from jax.experimental import pallas as pl
from jax.experimental.pallas import tpu as pltpu
from jax.experimental.pallas import tpu_sc as plsc

BATCH = 4
ROWS = 8192
D = 1024

NC = 2
NS = 16
NW = NC * NS
RPW = ROWS // NW
C = 64
NCHUNKS = RPW // C

_mesh = plsc.VectorSubcoreMesh(core_axis_name="c", subcore_axis_name="s")


@functools.partial(
    pl.kernel,
    mesh=_mesh,
    out_type=jax.ShapeDtypeStruct((BATCH * ROWS, D), jnp.float32),
    scratch_types=[
        pltpu.VMEM((C, D), jnp.float32),
        pltpu.SemaphoreType.DMA,
    ],
)
def _sc_copy(w_hbm, out_hbm, buf, ssem):
    wid = lax.axis_index("s") * NC + lax.axis_index("c")
    base = wid * RPW

    for ci in range(NCHUNKS):
        r0 = base + ci * C
        pltpu.sync_copy(w_hbm.at[pl.ds(r0, C)], buf)
        for b in range(BATCH):
            pltpu.make_async_copy(
                buf, out_hbm.at[pl.ds(b * ROWS + r0, C)], ssem).start()
        for b in range(BATCH):
            pltpu.make_async_copy(
                buf, out_hbm.at[pl.ds(b * ROWS + r0, C)], ssem).wait()


def kernel(input_ids, weight):
    del input_ids
    out = _sc_copy(weight)
    return out.reshape(BATCH, ROWS, D)


# speedup vs baseline: 1.0179x; 1.0003x over previous
import functools

import jax
import jax.numpy as jnp
from jax import lax
from jax.experimental import pallas as pl
from jax.experimental.pallas import tpu as pltpu
from jax.experimental.pallas import tpu_sc as plsc

BATCH = 4
ROWS = 8192
D = 1024

NC = 2
NS = 16
NW = NC * NS
RPW = ROWS // NW
C = 64
NCHUNKS = RPW // C

_mesh = plsc.VectorSubcoreMesh(core_axis_name="c", subcore_axis_name="s")


@functools.partial(
    pl.kernel,
    mesh=_mesh,
    out_type=jax.ShapeDtypeStruct((BATCH * ROWS, D), jnp.float32),
    scratch_types=[
        pltpu.VMEM((C, D), jnp.float32),
        pltpu.SemaphoreType.DMA,
    ],
)
def _sc_copy(w_hbm, out_hbm, buf, ssem):
    wid = lax.axis_index("c") * NS + lax.axis_index("s")
    base = wid * RPW

    for ci in range(NCHUNKS):
        r0 = base + ci * C
        pltpu.sync_copy(w_hbm.at[pl.ds(r0, C)], buf)
        for b in range(BATCH):
            pltpu.make_async_copy(
                buf, out_hbm.at[pl.ds(b * ROWS + r0, C)], ssem).start()
        for b in range(BATCH):
            pltpu.make_async_copy(
                buf, out_hbm.at[pl.ds(b * ROWS + r0, C)], ssem).wait()


def kernel(input_ids, weight):
    del input_ids
    out = _sc_copy(weight)
    return out.reshape(BATCH, ROWS, D)
